# baseline (speedup 1.0000x reference)
import functools

import jax
import jax.numpy as jnp
from jax import lax
from jax.experimental import pallas as pl
from jax.experimental.pallas import tpu as pltpu
from jax.experimental.pallas import tpu_sc as plsc

NC, NS, LANES = 2, 16, 16
NW = NC * NS
CH_ROWS = 16


def kernel(x, pe_table):
    bs, t, d = x.shape
    chunk = CH_ROWS * d
    rows_per_w = t // NW
    nchunks = rows_per_w // CH_ROWS
    assert rows_per_w * NW == t and nchunks * CH_ROWS == rows_per_w

    x_flat = x.reshape(bs, t * d)
    pe_flat = pe_table.reshape(-1)

    mesh = plsc.VectorSubcoreMesh(
        core_axis_name="c", subcore_axis_name="s", num_cores=NC, num_subcores=NS
    )

    @functools.partial(
        pl.kernel,
        out_type=jax.ShapeDtypeStruct((bs, t * d), jnp.float32),
        mesh=mesh,
        scratch_types=[
            pltpu.VMEM((chunk,), jnp.float32),
            pltpu.VMEM((bs, chunk), jnp.float32),
        ],
    )
    def pe_add(x_hbm, pe_hbm, out_hbm, pe_v, x_v):
        wid = lax.axis_index("s") * NC + lax.axis_index("c")
        base = wid * nchunks * chunk

        def chunk_body(c, _):
            off = base + c * chunk
            pltpu.sync_copy(pe_hbm.at[pl.ds(off, chunk)], pe_v)
            for b in range(bs):
                pltpu.sync_copy(x_hbm.at[b, pl.ds(off, chunk)], x_v.at[b])

            def slice_body(i, _):
                s = pl.ds(i * LANES, LANES)
                pv = pe_v[s]
                for b in range(bs):
                    x_v[b, s] = x_v[b, s] + pv
                return 0

            lax.fori_loop(0, chunk // LANES, slice_body, 0, unroll=4)
            for b in range(bs):
                pltpu.sync_copy(x_v.at[b], out_hbm.at[b, pl.ds(off, chunk)])
            return 0

        lax.fori_loop(0, nchunks, chunk_body, 0)

    out = pe_add(x_flat, pe_flat)
    return out.reshape(bs, t, d)


# speedup vs baseline: 1.2312x; 1.2312x over previous
import functools

import jax
import jax.numpy as jnp
from jax import lax
from jax.experimental import pallas as pl
from jax.experimental.pallas import tpu as pltpu
from jax.experimental.pallas import tpu_sc as plsc

NC, NS, LANES = 2, 16, 16
NW = NC * NS
CH_ROWS = 8


def kernel(x, pe_table):
    bs, t, d = x.shape
    chunk = CH_ROWS * d
    rows_per_w = t // NW
    nchunks = rows_per_w // CH_ROWS
    assert rows_per_w * NW == t and nchunks * CH_ROWS == rows_per_w

    x_flat = x.reshape(bs, t * d)
    pe_flat = pe_table.reshape(-1)

    mesh = plsc.VectorSubcoreMesh(
        core_axis_name="c", subcore_axis_name="s", num_cores=NC, num_subcores=NS
    )

    @functools.partial(
        pl.kernel,
        out_type=jax.ShapeDtypeStruct((bs, t * d), jnp.float32),
        mesh=mesh,
        scratch_types=[
            pltpu.VMEM((2, chunk), jnp.float32),
            pltpu.VMEM((2, bs, chunk), jnp.float32),
            pltpu.SemaphoreType.DMA((2,)),
            pltpu.SemaphoreType.DMA((2,)),
        ],
    )
    def pe_add(x_hbm, pe_hbm, out_hbm, pe_v, x_v, load_sem, store_sem):
        wid = lax.axis_index("s") * NC + lax.axis_index("c")
        base = wid * nchunks * chunk

        def start_loads(c, s):
            off = base + c * chunk
            return [
                pltpu.async_copy(
                    pe_hbm.at[pl.ds(off, chunk)], pe_v.at[s], load_sem.at[s]
                ),
                pltpu.async_copy(
                    x_hbm.at[:, pl.ds(off, chunk)], x_v.at[s], load_sem.at[s]
                ),
            ]

        def compute(s):
            def slice_body(i, _):
                sl = pl.ds(i * LANES, LANES)
                pv = pe_v[s, sl]
                for b in range(bs):
                    x_v[s, b, sl] = x_v[s, b, sl] + pv
                return 0

            lax.fori_loop(0, chunk // LANES, slice_body, 0, unroll=4)

        def start_store(c, s):
            off = base + c * chunk
            return pltpu.async_copy(
                x_v.at[s], out_hbm.at[:, pl.ds(off, chunk)], store_sem.at[s]
            )

        store_h = [None, None]
        load_h = start_loads(0, 0)
        next_load_h = None
        for c in range(nchunks):
            s = c % 2
            ns = (c + 1) % 2
            if c + 1 < nchunks:
                if store_h[ns] is not None:
                    store_h[ns].wait()
                    store_h[ns] = None
                next_load_h = start_loads(c + 1, ns)
            for h in load_h:
                h.wait()
            compute(s)
            store_h[s] = start_store(c, s)
            load_h = next_load_h
        for hh in store_h:
            if hh is not None:
                hh.wait()

    out = pe_add(x_flat, pe_flat)
    return out.reshape(bs, t, d)


# speedup vs baseline: 1.6962x; 1.3777x over previous
import functools

import jax
import jax.numpy as jnp
from jax import lax
from jax.experimental import pallas as pl
from jax.experimental.pallas import tpu as pltpu
from jax.experimental.pallas import tpu_sc as plsc

NC, NS, LANES = 2, 16, 16
NW = NC * NS
CH_ROWS = 8


def kernel(x, pe_table):
    bs, t, d = x.shape
    chunk = CH_ROWS * d
    rows_per_w = t // NW
    nchunks = rows_per_w // CH_ROWS
    assert rows_per_w * NW == t and nchunks * CH_ROWS == rows_per_w

    x_flat = x.reshape(bs, t * d)
    pe_flat = pe_table.reshape(-1)

    mesh = plsc.VectorSubcoreMesh(
        core_axis_name="c", subcore_axis_name="s", num_cores=NC, num_subcores=NS
    )

    @functools.partial(
        pl.kernel,
        out_type=jax.ShapeDtypeStruct((bs, t * d), jnp.float32),
        mesh=mesh,
        scratch_types=[
            pltpu.VMEM((2, chunk), jnp.float32),
            pltpu.VMEM((2, bs, chunk), jnp.float32),
            pltpu.SemaphoreType.DMA((2,)),
            pltpu.SemaphoreType.DMA((2,)),
        ],
    )
    def pe_add(x_hbm, pe_hbm, out_hbm, pe_v, x_v, load_sem, store_sem):
        wid = lax.axis_index("s") * NC + lax.axis_index("c")
        base = wid * nchunks * chunk

        def start_loads(c, s):
            off = base + c * chunk
            return [
                pltpu.async_copy(
                    pe_hbm.at[pl.ds(off, chunk)], pe_v.at[s], load_sem.at[s]
                ),
                pltpu.async_copy(
                    x_hbm.at[:, pl.ds(off, chunk)], x_v.at[s], load_sem.at[s]
                ),
            ]

        def compute(s):
            @plsc.parallel_loop(0, chunk, step=LANES, unroll=8)
            def slice_body(i):
                sl = pl.ds(i, LANES)
                pv = pe_v[s, sl]
                for b in range(bs):
                    x_v[s, b, sl] = x_v[s, b, sl] + pv

        def start_store(c, s):
            off = base + c * chunk
            return pltpu.async_copy(
                x_v.at[s], out_hbm.at[:, pl.ds(off, chunk)], store_sem.at[s]
            )

        store_h = [None, None]
        load_h = start_loads(0, 0)
        next_load_h = None
        for c in range(nchunks):
            s = c % 2
            ns = (c + 1) % 2
            if c + 1 < nchunks:
                if store_h[ns] is not None:
                    store_h[ns].wait()
                    store_h[ns] = None
                next_load_h = start_loads(c + 1, ns)
            for h in load_h:
                h.wait()
            compute(s)
            store_h[s] = start_store(c, s)
            load_h = next_load_h
        for hh in store_h:
            if hh is not None:
                hh.wait()

    out = pe_add(x_flat, pe_flat)
    return out.reshape(bs, t, d)


# speedup vs baseline: 4.5774x; 2.6986x over previous
import functools

import jax
import jax.numpy as jnp
from jax import lax
from jax.experimental import pallas as pl
from jax.experimental.pallas import tpu as pltpu
from jax.experimental.pallas import tpu_sc as plsc

NC, NS, LANES = 2, 16, 16
NW = NC * NS
CH_ROWS = 8


def kernel(x, pe_table):
    bs, t, d = x.shape
    rows_per_w = t // NW
    nchunks = rows_per_w // CH_ROWS
    assert rows_per_w * NW == t and nchunks * CH_ROWS == rows_per_w

    mesh = plsc.VectorSubcoreMesh(
        core_axis_name="c", subcore_axis_name="s", num_cores=NC, num_subcores=NS
    )

    @functools.partial(
        pl.kernel,
        out_type=jax.ShapeDtypeStruct((bs, t, d), jnp.float32),
        mesh=mesh,
        scratch_types=[
            pltpu.VMEM((2, CH_ROWS, d), jnp.float32),
            pltpu.VMEM((2, bs, CH_ROWS, d), jnp.float32),
            pltpu.SemaphoreType.DMA((2,)),
            pltpu.SemaphoreType.DMA((2,)),
        ],
    )
    def pe_add(x_hbm, pe_hbm, out_hbm, pe_v, x_v, load_sem, store_sem):
        wid = lax.axis_index("s") * NC + lax.axis_index("c")
        row0 = wid * rows_per_w

        def start_loads(c, s):
            rows = pl.ds(row0 + c * CH_ROWS, CH_ROWS)
            return [
                pltpu.async_copy(pe_hbm.at[rows, :], pe_v.at[s], load_sem.at[s]),
                pltpu.async_copy(x_hbm.at[:, rows, :], x_v.at[s], load_sem.at[s]),
            ]

        def compute(s):
            @plsc.parallel_loop(0, d, step=LANES, unroll=2)
            def slice_body(i):
                sl = pl.ds(i, LANES)
                for r in range(CH_ROWS):
                    pv = pe_v[s, r, sl]
                    for b in range(bs):
                        x_v[s, b, r, sl] = x_v[s, b, r, sl] + pv

        def start_store(c, s):
            rows = pl.ds(row0 + c * CH_ROWS, CH_ROWS)
            return pltpu.async_copy(x_v.at[s], out_hbm.at[:, rows, :], store_sem.at[s])

        rows0 = pl.ds(row0, CH_ROWS)

        def wait_loads(s):
            pltpu.make_async_copy(
                pe_hbm.at[rows0, :], pe_v.at[s], load_sem.at[s]
            ).wait()
            pltpu.make_async_copy(
                x_hbm.at[:, rows0, :], x_v.at[s], load_sem.at[s]
            ).wait()

        def wait_store(s):
            pltpu.make_async_copy(
                x_v.at[s], out_hbm.at[:, rows0, :], store_sem.at[s]
            ).wait()

        start_loads(0, 0)

        def pair_body(k, _):
            c0 = 2 * k

            @pl.when(k > 0)
            def _():
                wait_store(1)

            start_loads(c0 + 1, 1)
            wait_loads(0)
            compute(0)
            start_store(c0, 0)

            wait_store(0)

            @pl.when(c0 + 2 < nchunks)
            def _():
                start_loads(c0 + 2, 0)

            wait_loads(1)
            compute(1)
            start_store(c0 + 1, 1)
            return 0

        lax.fori_loop(0, nchunks // 2, pair_body, 0)
        wait_store(1)

    return pe_add(x, pe_table)


# speedup vs baseline: 4.6973x; 1.0262x over previous
import functools

import jax
import jax.numpy as jnp
from jax import lax
from jax.experimental import pallas as pl
from jax.experimental.pallas import tpu as pltpu
from jax.experimental.pallas import tpu_sc as plsc

NC, NS, LANES = 2, 16, 16
NW = NC * NS
CH_ROWS = 4
RING = 4


def kernel(x, pe_table):
    bs, t, d = x.shape
    rows_per_w = t // NW
    nchunks = rows_per_w // CH_ROWS
    ngroups = nchunks // RING
    assert rows_per_w * NW == t and ngroups * RING == nchunks

    mesh = plsc.VectorSubcoreMesh(
        core_axis_name="c", subcore_axis_name="s", num_cores=NC, num_subcores=NS
    )

    @functools.partial(
        pl.kernel,
        out_type=jax.ShapeDtypeStruct((bs, t, d), jnp.float32),
        mesh=mesh,
        scratch_types=[
            pltpu.VMEM((RING, CH_ROWS, d), jnp.float32),
            pltpu.VMEM((RING, bs, CH_ROWS, d), jnp.float32),
            pltpu.SemaphoreType.DMA((RING,)),
            pltpu.SemaphoreType.DMA((RING,)),
        ],
    )
    def pe_add(x_hbm, pe_hbm, out_hbm, pe_v, x_v, load_sem, store_sem):
        wid = lax.axis_index("s") * NC + lax.axis_index("c")
        row0 = wid * rows_per_w

        def start_loads(c, s):
            rows = pl.ds(row0 + c * CH_ROWS, CH_ROWS)
            pltpu.async_copy(pe_hbm.at[rows, :], pe_v.at[s], load_sem.at[s])
            pltpu.async_copy(x_hbm.at[:, rows, :], x_v.at[s], load_sem.at[s])

        def start_store(c, s):
            rows = pl.ds(row0 + c * CH_ROWS, CH_ROWS)
            pltpu.async_copy(x_v.at[s], out_hbm.at[:, rows, :], store_sem.at[s])

        rows0 = pl.ds(row0, CH_ROWS)

        def wait_loads(s):
            pltpu.make_async_copy(
                pe_hbm.at[rows0, :], pe_v.at[s], load_sem.at[s]
            ).wait()
            pltpu.make_async_copy(
                x_hbm.at[:, rows0, :], x_v.at[s], load_sem.at[s]
            ).wait()

        def wait_store(s):
            pltpu.make_async_copy(
                x_v.at[s], out_hbm.at[:, rows0, :], store_sem.at[s]
            ).wait()

        def compute(s):
            @plsc.parallel_loop(0, d, step=LANES, unroll=2)
            def slice_body(i):
                sl = pl.ds(i, LANES)
                for r in range(CH_ROWS):
                    pv = pe_v[s, r, sl]
                    for b in range(bs):
                        x_v[s, b, r, sl] = x_v[s, b, r, sl] + pv

        start_loads(0, 0)
        start_loads(1, 1)

        def group_body(k, _):
            for j in range(RING):
                c = RING * k + j
                s = j
                tgt = (j + 2) % RING
                if j < 2:
                    @pl.when(k > 0)
                    def _():
                        wait_store(tgt)

                    start_loads(c + 2, tgt)
                else:
                    wait_store(tgt)

                    @pl.when(k + 1 < ngroups)
                    def _():
                        start_loads(c + 2, tgt)

                wait_loads(s)
                compute(s)
                start_store(c, s)
            return 0

        lax.fori_loop(0, ngroups, group_body, 0)
        wait_store((nchunks - 2) % RING)
        wait_store((nchunks - 1) % RING)

    return pe_add(x, pe_table)


# speedup vs baseline: 4.7376x; 1.0086x over previous
import functools

import jax
import jax.numpy as jnp
from jax import lax
from jax.experimental import pallas as pl
from jax.experimental.pallas import tpu as pltpu
from jax.experimental.pallas import tpu_sc as plsc

NC, NS, LANES = 2, 16, 16
NW = NC * NS
CH_ROWS = 2
RING = 8
PF = 4


def kernel(x, pe_table):
    bs, t, d = x.shape
    rows_per_w = t // NW
    nchunks = rows_per_w // CH_ROWS
    ngroups = nchunks // RING
    assert rows_per_w * NW == t and ngroups * RING == nchunks

    mesh = plsc.VectorSubcoreMesh(
        core_axis_name="c", subcore_axis_name="s", num_cores=NC, num_subcores=NS
    )

    @functools.partial(
        pl.kernel,
        out_type=jax.ShapeDtypeStruct((bs, t, d), jnp.float32),
        mesh=mesh,
        scratch_types=[
            pltpu.VMEM((RING, CH_ROWS, d), jnp.float32),
            pltpu.VMEM((RING, bs, CH_ROWS, d), jnp.float32),
            pltpu.SemaphoreType.DMA((RING,)),
            pltpu.SemaphoreType.DMA((RING,)),
        ],
    )
    def pe_add(x_hbm, pe_hbm, out_hbm, pe_v, x_v, load_sem, store_sem):
        wid = lax.axis_index("s") * NC + lax.axis_index("c")
        row0 = wid * rows_per_w

        def start_loads(c, s):
            rows = pl.ds(row0 + c * CH_ROWS, CH_ROWS)
            pltpu.async_copy(pe_hbm.at[rows, :], pe_v.at[s], load_sem.at[s])
            pltpu.async_copy(x_hbm.at[:, rows, :], x_v.at[s], load_sem.at[s])

        def start_store(c, s):
            rows = pl.ds(row0 + c * CH_ROWS, CH_ROWS)
            pltpu.async_copy(x_v.at[s], out_hbm.at[:, rows, :], store_sem.at[s])

        rows0 = pl.ds(row0, CH_ROWS)

        def wait_loads(s):
            pltpu.make_async_copy(
                pe_hbm.at[rows0, :], pe_v.at[s], load_sem.at[s]
            ).wait()
            pltpu.make_async_copy(
                x_hbm.at[:, rows0, :], x_v.at[s], load_sem.at[s]
            ).wait()

        def wait_store(s):
            pltpu.make_async_copy(
                x_v.at[s], out_hbm.at[:, rows0, :], store_sem.at[s]
            ).wait()

        def compute(s):
            @plsc.parallel_loop(0, d, step=LANES, unroll=2)
            def slice_body(i):
                sl = pl.ds(i, LANES)
                for r in range(CH_ROWS):
                    pv = pe_v[s, r, sl]
                    for b in range(bs):
                        x_v[s, b, r, sl] = x_v[s, b, r, sl] + pv

        for i in range(PF):
            start_loads(i, i)

        def group_body(k, _):
            for j in range(RING):
                c = RING * k + j
                tgt = (j + PF) % RING
                if j < RING - PF:
                    @pl.when(k > 0)
                    def _():
                        wait_store(tgt)

                    start_loads(c + PF, tgt)
                else:
                    wait_store(tgt)

                    @pl.when(k + 1 < ngroups)
                    def _():
                        start_loads(c + PF, tgt)

                wait_loads(j)
                compute(j)
                start_store(c, j)
            return 0

        lax.fori_loop(0, ngroups, group_body, 0)
        for i in range(PF):
            wait_store((nchunks - PF + i) % RING)

    return pe_add(x, pe_table)
